# baseline (speedup 1.0000x reference)
import functools

import jax
import jax.numpy as jnp
from jax import lax
from jax.experimental import pallas as pl
from jax.experimental.pallas import tpu as pltpu
from jax.experimental.pallas import tpu_sc as plsc

N = 10000
D_IN = 128
D_MID = 128
D_BOT = 64

NC = 2
NS = 16
NW = NC * NS

N_PAD = 10240
ROWS_PER_TILE = N_PAD // NS

CH = 88
E_EDGES = 320000
E_ALL = E_EDGES + N
CHUNKS_PER_TILE = 120
EDGES_PER_TILE = CHUNKS_PER_TILE * CH
E_PAD = EDGES_PER_TILE * NW

DEGW = 128
BN = 1024
GRID = N_PAD // BN


NBUF = 4
NIDX = 8
GROUPS = CHUNKS_PER_TILE // 3
GROUPS6 = CHUNKS_PER_TILE // NIDX


@functools.lru_cache(maxsize=None)
def _make_deg_scatter():
    mesh = plsc.VectorSubcoreMesh(core_axis_name="c", subcore_axis_name="s")

    @functools.partial(
        pl.kernel,
        out_type=jax.ShapeDtypeStruct((NC, N_PAD, DEGW), jnp.float32),
        mesh=mesh,
        scratch_types=[
            pltpu.VMEM((CHUNKS_PER_TILE, CH), jnp.int32),
            pltpu.VMEM((CH, DEGW), jnp.float32),
            pltpu.VMEM_SHARED((N_PAD, DEGW), jnp.float32),
            pltpu.SemaphoreType.DMA,
            pltpu.SemaphoreType.DMA,
            pltpu.SemaphoreType.DMA,
        ],
    )
    def degk(dst_hbm, ones_hbm, zrows_hbm, out_hbm, didx, ones_v, acc,
             ss0, ss1, ss2):
        c = lax.axis_index("c")
        s = lax.axis_index("s")
        tid = c * NS + s
        sss = [ss0, ss1, ss2]

        pltpu.sync_copy(ones_hbm, ones_v)
        pltpu.sync_copy(dst_hbm.at[tid], didx)
        pltpu.sync_copy(zrows_hbm, acc.at[pl.ds(s * ROWS_PER_TILE, ROWS_PER_TILE)])
        plsc.subcore_barrier()

        def group(g, carry):
            for b in range(3):
                k = g * 3 + b
                pltpu.async_copy(ones_v, acc.at[didx.at[k]], sss[b], add=True)

                @pl.when(k >= 2)
                def _():
                    bp = (b + 1) % 3
                    pltpu.make_async_copy(
                        ones_v, acc.at[didx.at[k - 2]], sss[bp]).wait()
            return carry

        lax.fori_loop(0, GROUPS, group, 0)
        for k in (CHUNKS_PER_TILE - 2, CHUNKS_PER_TILE - 1):
            pltpu.make_async_copy(ones_v, acc.at[didx.at[k]],
                                  sss[k % 3]).wait()
        plsc.subcore_barrier()
        pltpu.sync_copy(
            acc.at[pl.ds(s * ROWS_PER_TILE, ROWS_PER_TILE)],
            out_hbm.at[c, pl.ds(s * ROWS_PER_TILE, ROWS_PER_TILE)],
        )

    return degk


@functools.lru_cache(maxsize=None)
def _make_scatter(d: int):
    mesh = plsc.VectorSubcoreMesh(core_axis_name="c", subcore_axis_name="s")

    @functools.partial(
        pl.kernel,
        out_type=jax.ShapeDtypeStruct((NC, N_PAD, d), jnp.float32),
        mesh=mesh,
        scratch_types=[
            pltpu.VMEM((NIDX, CH), jnp.int32),
            pltpu.VMEM((NIDX, CH), jnp.int32),
            pltpu.VMEM((NBUF, CH, d), jnp.float32),
            pltpu.VMEM_SHARED((N_PAD, d), jnp.float32),
            [pltpu.SemaphoreType.DMA] * NBUF,
            [pltpu.SemaphoreType.DMA] * NBUF,
            [pltpu.SemaphoreType.DMA] * NIDX,
        ],
    )
    def scat(g_hbm, src_hbm, dst_hbm, zrows_hbm, out_hbm,
             sidx, didx, rows, acc, sgs, sss, sis):
        c = lax.axis_index("c")
        s = lax.axis_index("s")
        tid = c * NS + s

        def idx_copy(k, j, sync):
            if sync:
                pltpu.sync_copy(src_hbm.at[tid, k], sidx.at[j])
                pltpu.sync_copy(dst_hbm.at[tid, k], didx.at[j])
            else:
                pltpu.async_copy(src_hbm.at[tid, k], sidx.at[j], sis[j])
                pltpu.async_copy(dst_hbm.at[tid, k], didx.at[j], sis[j])

        def idx_wait(k, j):
            pltpu.make_async_copy(src_hbm.at[tid, k], sidx.at[j], sis[j]).wait()
            pltpu.make_async_copy(dst_hbm.at[tid, k], didx.at[j], sis[j]).wait()

        def issue_gather(j, b):
            pltpu.async_copy(g_hbm.at[sidx.at[j]], rows.at[b], sgs[b])

        idx_copy(0, 0, True)
        idx_copy(1, 1, True)
        idx_copy(2, 2, True)
        issue_gather(0, 0)
        issue_gather(1, 1)
        issue_gather(2, 2)
        idx_copy(3, 3, False)
        idx_copy(4, 4, False)
        pltpu.sync_copy(zrows_hbm, acc.at[pl.ds(s * ROWS_PER_TILE, ROWS_PER_TILE)])
        plsc.subcore_barrier()

        def group(g, carry):
            for u in range(NIDX):
                k = g * NIDX + u
                b = u % NBUF
                j = u
                jn = (u + 3) % NIDX
                jf = (u + 5) % NIDX
                bn = (b + 3) % NBUF
                pltpu.make_async_copy(
                    g_hbm.at[sidx.at[j]], rows.at[b], sgs[b]).wait()
                pltpu.async_copy(rows.at[b], acc.at[didx.at[j]], sss[b],
                                 add=True)

                jp = (u - 1) % NIDX

                @pl.when(k >= 1)
                def _():
                    pltpu.make_async_copy(
                        rows.at[bn], acc.at[didx.at[jp]], sss[bn]).wait()

                @pl.when(k + 3 < CHUNKS_PER_TILE)
                def _():
                    idx_wait(k + 3, jn)
                    issue_gather(jn, bn)

                @pl.when(k + 5 < CHUNKS_PER_TILE)
                def _():
                    idx_copy(k + 5, jf, False)
            return carry

        lax.fori_loop(0, GROUPS6, group, 0)
        kl = CHUNKS_PER_TILE - 1
        pltpu.make_async_copy(
            rows.at[kl % NBUF], acc.at[didx.at[kl % NIDX]],
            sss[kl % NBUF]).wait()
        plsc.subcore_barrier()

        pltpu.sync_copy(
            acc.at[pl.ds(s * ROWS_PER_TILE, ROWS_PER_TILE)],
            out_hbm.at[c, pl.ds(s * ROWS_PER_TILE, ROWS_PER_TILE)],
        )

    return scat


def _t_first(dacc, x_pad, w1):
    def body(dacc_ref, x_ref, w_ref, g_ref, dinv_ref):
        a = dacc_ref[...]
        deg = jnp.max(a[0] + a[1], axis=1, keepdims=True)
        dinv = lax.rsqrt(jnp.maximum(deg, 1.0))
        dinv_ref[...] = dinv
        g_ref[...] = jnp.dot(x_ref[...] * dinv, w_ref[...],
                             preferred_element_type=jnp.float32)

    return pl.pallas_call(
        body,
        grid=(GRID,),
        in_specs=[
            pl.BlockSpec((NC, BN, DEGW), lambda i: (0, i, 0)),
            pl.BlockSpec((BN, D_IN), lambda i: (i, 0)),
            pl.BlockSpec((D_IN, D_MID), lambda i: (0, 0)),
        ],
        out_specs=[
            pl.BlockSpec((BN, D_MID), lambda i: (i, 0)),
            pl.BlockSpec((BN, 1), lambda i: (i, 0)),
        ],
        out_shape=[
            jax.ShapeDtypeStruct((N_PAD, D_MID), jnp.float32),
            jax.ShapeDtypeStruct((N_PAD, 1), jnp.float32),
        ],
    )(dacc, x_pad, w1)


def _t_mid(acc, bias, dinv, w, relu: bool):
    dp = acc.shape[2]
    dn = w.shape[1]

    def body(acc_ref, b_ref, dinv_ref, w_ref, g_ref):
        a = acc_ref[...]
        dinv = dinv_ref[...]
        conv = (a[0] + a[1]) * dinv + b_ref[...]
        if relu:
            conv = jnp.maximum(conv, 0.0)
        g_ref[...] = jnp.dot(conv * dinv, w_ref[...],
                             preferred_element_type=jnp.float32)

    return pl.pallas_call(
        body,
        grid=(GRID,),
        in_specs=[
            pl.BlockSpec((NC, BN, dp), lambda i: (0, i, 0)),
            pl.BlockSpec((1, dp), lambda i: (0, 0)),
            pl.BlockSpec((BN, 1), lambda i: (i, 0)),
            pl.BlockSpec((dp, dn), lambda i: (0, 0)),
        ],
        out_specs=pl.BlockSpec((BN, dn), lambda i: (i, 0)),
        out_shape=jax.ShapeDtypeStruct((N_PAD, dn), jnp.float32),
    )(acc, bias, dinv, w)


def _t_final(acc, bias, dinv):
    dp = acc.shape[2]

    def body(acc_ref, b_ref, dinv_ref, o_ref):
        a = acc_ref[...]
        o_ref[...] = (a[0] + a[1]) * dinv_ref[...] + b_ref[...]

    return pl.pallas_call(
        body,
        grid=(GRID,),
        in_specs=[
            pl.BlockSpec((NC, BN, dp), lambda i: (0, i, 0)),
            pl.BlockSpec((1, dp), lambda i: (0, 0)),
            pl.BlockSpec((BN, 1), lambda i: (i, 0)),
        ],
        out_specs=pl.BlockSpec((BN, dp), lambda i: (i, 0)),
        out_shape=jax.ShapeDtypeStruct((N_PAD, dp), jnp.float32),
    )(acc, bias, dinv)


def kernel(x, edge_index, W1, b1, W2, b2, W3, b3, W4, b4):
    src = edge_index[0].astype(jnp.int32)
    dst = edge_index[1].astype(jnp.int32)
    loops = jnp.arange(N, dtype=jnp.int32)
    padv = N + jnp.arange(E_PAD - E_ALL, dtype=jnp.int32) % (N_PAD - N)
    src_cat = jnp.concatenate([src, loops, padv])
    dst_cat = jnp.concatenate([dst, loops, padv])
    order = jnp.argsort(src_cat)
    src_all = src_cat[order].reshape(NW, CHUNKS_PER_TILE, CH)
    dst_all = dst_cat[order].reshape(NW, CHUNKS_PER_TILE, CH)

    x_pad = jnp.zeros((N_PAD, D_IN), jnp.float32).at[:N].set(x)
    ones128 = jnp.ones((CH, DEGW), jnp.float32)
    zdeg = jnp.zeros((ROWS_PER_TILE, DEGW), jnp.float32)
    z128 = jnp.zeros((ROWS_PER_TILE, 128), jnp.float32)

    W2p = jnp.zeros((D_MID, 128), jnp.float32).at[:, :D_BOT].set(W2)
    W3p = jnp.zeros((128, D_MID), jnp.float32).at[:D_BOT, :].set(W3)
    b2p = jnp.zeros((128,), jnp.float32).at[:D_BOT].set(b2)

    b1r = b1.reshape(1, -1)
    b2r = b2p.reshape(1, -1)
    b3r = b3.reshape(1, -1)
    b4r = b4.reshape(1, -1)

    dacc = _make_deg_scatter()(dst_all, ones128, zdeg)

    g1, dinv = _t_first(dacc, x_pad, W1)
    acc1 = _make_scatter(128)(g1, src_all, dst_all, z128)
    g2 = _t_mid(acc1, b1r, dinv, W2p, relu=True)
    acc2 = _make_scatter(128)(g2, src_all, dst_all, z128)
    g3 = _t_mid(acc2, b2r, dinv, W3p, relu=False)
    acc3 = _make_scatter(128)(g3, src_all, dst_all, z128)
    g4 = _t_mid(acc3, b3r, dinv, W4, relu=True)
    acc4 = _make_scatter(128)(g4, src_all, dst_all, z128)
    out = _t_final(acc4, b4r, dinv)
    return out[:N]


# speedup vs baseline: 2.6073x; 2.6073x over previous
import functools

import jax
import jax.numpy as jnp
from jax import lax
from jax.experimental import pallas as pl
from jax.experimental.pallas import tpu as pltpu
from jax.experimental.pallas import tpu_sc as plsc

N = 10000
D_IN = 128
D_MID = 128
D_BOT = 64

NC = 2
NS = 16
NW = NC * NS

N_PAD = 10240
ROWS_PER_TILE = N_PAD // NS

CH = 88
E_EDGES = 320000
E_ALL = E_EDGES + N
CHUNKS_PER_TILE = 120
EDGES_PER_TILE = CHUNKS_PER_TILE * CH
E_PAD = EDGES_PER_TILE * NW

BN = 1024
GRID = N_PAD // BN


NBUF = 4
NIDX = 8
GROUPS = CHUNKS_PER_TILE // 3
GROUPS6 = CHUNKS_PER_TILE // NIDX


@functools.lru_cache(maxsize=None)
def _make_deg_scatter():
    mesh = plsc.VectorSubcoreMesh(core_axis_name="c", subcore_axis_name="s")

    @functools.partial(
        pl.kernel,
        out_type=jax.ShapeDtypeStruct((NC, N_PAD, 16), jnp.float32),
        mesh=mesh,
        compiler_params=pltpu.CompilerParams(use_tc_tiling_on_sc=False),
        scratch_types=[
            pltpu.VMEM((CHUNKS_PER_TILE, CH), jnp.int32),
            pltpu.VMEM((CH, 16), jnp.float32),
            pltpu.VMEM_SHARED((N_PAD, 16), jnp.float32),
            pltpu.SemaphoreType.DMA,
            pltpu.SemaphoreType.DMA,
            pltpu.SemaphoreType.DMA,
        ],
    )
    def degk(dst_hbm, ones_hbm, zrows_hbm, out_hbm, didx, ones_v, acc,
             ss0, ss1, ss2):
        c = lax.axis_index("c")
        s = lax.axis_index("s")
        tid = c * NS + s
        sss = [ss0, ss1, ss2]

        pltpu.sync_copy(ones_hbm, ones_v)
        pltpu.sync_copy(dst_hbm.at[tid], didx)
        pltpu.sync_copy(zrows_hbm, acc.at[pl.ds(s * ROWS_PER_TILE, ROWS_PER_TILE)])
        plsc.subcore_barrier()

        def group(g, carry):
            for b in range(3):
                k = g * 3 + b
                pltpu.async_copy(ones_v, acc.at[didx.at[k]], sss[b], add=True)

                @pl.when(k >= 2)
                def _():
                    bp = (b + 1) % 3
                    pltpu.make_async_copy(
                        ones_v, acc.at[didx.at[k - 2]], sss[bp]).wait()
            return carry

        lax.fori_loop(0, GROUPS, group, 0)
        for k in (CHUNKS_PER_TILE - 2, CHUNKS_PER_TILE - 1):
            pltpu.make_async_copy(ones_v, acc.at[didx.at[k]],
                                  sss[k % 3]).wait()
        plsc.subcore_barrier()
        pltpu.sync_copy(
            acc.at[pl.ds(s * ROWS_PER_TILE, ROWS_PER_TILE)],
            out_hbm.at[c, pl.ds(s * ROWS_PER_TILE, ROWS_PER_TILE)],
        )

    return degk


@functools.lru_cache(maxsize=None)
def _make_scatter(d: int):
    mesh = plsc.VectorSubcoreMesh(core_axis_name="c", subcore_axis_name="s")

    @functools.partial(
        pl.kernel,
        out_type=jax.ShapeDtypeStruct((NC, N_PAD, d), jnp.float32),
        mesh=mesh,
        scratch_types=[
            pltpu.VMEM((NIDX, CH), jnp.int32),
            pltpu.VMEM((NIDX, CH), jnp.int32),
            pltpu.VMEM((NBUF, CH, d), jnp.float32),
            pltpu.VMEM_SHARED((N_PAD, d), jnp.float32),
            [pltpu.SemaphoreType.DMA] * NBUF,
            [pltpu.SemaphoreType.DMA] * NBUF,
            [pltpu.SemaphoreType.DMA] * NIDX,
        ],
    )
    def scat(g_hbm, src_hbm, dst_hbm, zrows_hbm, out_hbm,
             sidx, didx, rows, acc, sgs, sss, sis):
        c = lax.axis_index("c")
        s = lax.axis_index("s")
        tid = c * NS + s

        def idx_copy(k, j, sync):
            if sync:
                pltpu.sync_copy(src_hbm.at[tid, k], sidx.at[j])
                pltpu.sync_copy(dst_hbm.at[tid, k], didx.at[j])
            else:
                pltpu.async_copy(src_hbm.at[tid, k], sidx.at[j], sis[j])
                pltpu.async_copy(dst_hbm.at[tid, k], didx.at[j], sis[j])

        def idx_wait(k, j):
            pltpu.make_async_copy(src_hbm.at[tid, k], sidx.at[j], sis[j]).wait()
            pltpu.make_async_copy(dst_hbm.at[tid, k], didx.at[j], sis[j]).wait()

        def issue_gather(j, b):
            pltpu.async_copy(g_hbm.at[sidx.at[j]], rows.at[b], sgs[b])

        idx_copy(0, 0, True)
        idx_copy(1, 1, True)
        idx_copy(2, 2, True)
        issue_gather(0, 0)
        issue_gather(1, 1)
        issue_gather(2, 2)
        idx_copy(3, 3, False)
        idx_copy(4, 4, False)
        pltpu.sync_copy(zrows_hbm, acc.at[pl.ds(s * ROWS_PER_TILE, ROWS_PER_TILE)])
        plsc.subcore_barrier()

        def group(g, carry):
            for u in range(NIDX):
                k = g * NIDX + u
                b = u % NBUF
                j = u
                jn = (u + 3) % NIDX
                jf = (u + 5) % NIDX
                bn = (b + 3) % NBUF
                pltpu.make_async_copy(
                    g_hbm.at[sidx.at[j]], rows.at[b], sgs[b]).wait()
                pltpu.async_copy(rows.at[b], acc.at[didx.at[j]], sss[b],
                                 add=True)

                jp = (u - 1) % NIDX

                @pl.when(k >= 1)
                def _():
                    pltpu.make_async_copy(
                        rows.at[bn], acc.at[didx.at[jp]], sss[bn]).wait()

                @pl.when(k + 3 < CHUNKS_PER_TILE)
                def _():
                    idx_wait(k + 3, jn)
                    issue_gather(jn, bn)

                @pl.when(k + 5 < CHUNKS_PER_TILE)
                def _():
                    idx_copy(k + 5, jf, False)
            return carry

        lax.fori_loop(0, GROUPS6, group, 0)
        kl = CHUNKS_PER_TILE - 1
        pltpu.make_async_copy(
            rows.at[kl % NBUF], acc.at[didx.at[kl % NIDX]],
            sss[kl % NBUF]).wait()
        plsc.subcore_barrier()

        pltpu.sync_copy(
            acc.at[pl.ds(s * ROWS_PER_TILE, ROWS_PER_TILE)],
            out_hbm.at[c, pl.ds(s * ROWS_PER_TILE, ROWS_PER_TILE)],
        )

    return scat


def _t_first(dacc, x_pad, w1):
    def body(dacc_ref, x_ref, w_ref, g_ref, dinv_ref):
        a = dacc_ref[...]
        deg = jnp.max(a[0] + a[1], axis=1, keepdims=True)
        dinv = lax.rsqrt(jnp.maximum(deg, 1.0))
        dinv_ref[...] = dinv
        g_ref[...] = jnp.dot(x_ref[...] * dinv, w_ref[...],
                             preferred_element_type=jnp.float32)

    return pl.pallas_call(
        body,
        grid=(GRID,),
        in_specs=[
            pl.BlockSpec((NC, BN, 16), lambda i: (0, i, 0)),
            pl.BlockSpec((BN, D_IN), lambda i: (i, 0)),
            pl.BlockSpec((D_IN, D_MID), lambda i: (0, 0)),
        ],
        out_specs=[
            pl.BlockSpec((BN, D_MID), lambda i: (i, 0)),
            pl.BlockSpec((BN, 1), lambda i: (i, 0)),
        ],
        out_shape=[
            jax.ShapeDtypeStruct((N_PAD, D_MID), jnp.float32),
            jax.ShapeDtypeStruct((N_PAD, 1), jnp.float32),
        ],
    )(dacc, x_pad, w1)


def _t_mid(acc, bias, dinv, w, relu: bool):
    dp = acc.shape[2]
    dn = w.shape[1]

    def body(acc_ref, b_ref, dinv_ref, w_ref, g_ref):
        a = acc_ref[...]
        dinv = dinv_ref[...]
        conv = (a[0] + a[1]) * dinv + b_ref[...]
        if relu:
            conv = jnp.maximum(conv, 0.0)
        g_ref[...] = jnp.dot(conv * dinv, w_ref[...],
                             preferred_element_type=jnp.float32)

    return pl.pallas_call(
        body,
        grid=(GRID,),
        in_specs=[
            pl.BlockSpec((NC, BN, dp), lambda i: (0, i, 0)),
            pl.BlockSpec((1, dp), lambda i: (0, 0)),
            pl.BlockSpec((BN, 1), lambda i: (i, 0)),
            pl.BlockSpec((dp, dn), lambda i: (0, 0)),
        ],
        out_specs=pl.BlockSpec((BN, dn), lambda i: (i, 0)),
        out_shape=jax.ShapeDtypeStruct((N_PAD, dn), jnp.float32),
    )(acc, bias, dinv, w)


def _t_final(acc, bias, dinv):
    dp = acc.shape[2]

    def body(acc_ref, b_ref, dinv_ref, o_ref):
        a = acc_ref[...]
        o_ref[...] = (a[0] + a[1]) * dinv_ref[...] + b_ref[...]

    return pl.pallas_call(
        body,
        grid=(GRID,),
        in_specs=[
            pl.BlockSpec((NC, BN, dp), lambda i: (0, i, 0)),
            pl.BlockSpec((1, dp), lambda i: (0, 0)),
            pl.BlockSpec((BN, 1), lambda i: (i, 0)),
        ],
        out_specs=pl.BlockSpec((BN, dp), lambda i: (i, 0)),
        out_shape=jax.ShapeDtypeStruct((N_PAD, dp), jnp.float32),
    )(acc, bias, dinv)


def kernel(x, edge_index, W1, b1, W2, b2, W3, b3, W4, b4):
    src = edge_index[0].astype(jnp.int32)
    dst = edge_index[1].astype(jnp.int32)
    loops = jnp.arange(N, dtype=jnp.int32)
    padv = N + jnp.arange(E_PAD - E_ALL, dtype=jnp.int32) % (N_PAD - N)
    src_all = jnp.concatenate([src, loops, padv]).reshape(NW, CHUNKS_PER_TILE, CH)
    dst_all = jnp.concatenate([dst, loops, padv]).reshape(NW, CHUNKS_PER_TILE, CH)

    x_pad = jnp.zeros((N_PAD, D_IN), jnp.float32).at[:N].set(x)
    ones128 = jnp.ones((CH, 16), jnp.float32)
    zdeg = jnp.zeros((ROWS_PER_TILE, 16), jnp.float32)
    z128 = jnp.zeros((ROWS_PER_TILE, 128), jnp.float32)

    W2p = jnp.zeros((D_MID, 128), jnp.float32).at[:, :D_BOT].set(W2)
    W3p = jnp.zeros((128, D_MID), jnp.float32).at[:D_BOT, :].set(W3)
    b2p = jnp.zeros((128,), jnp.float32).at[:D_BOT].set(b2)

    b1r = b1.reshape(1, -1)
    b2r = b2p.reshape(1, -1)
    b3r = b3.reshape(1, -1)
    b4r = b4.reshape(1, -1)

    dacc = _make_deg_scatter()(dst_all, ones128, zdeg)

    g1, dinv = _t_first(dacc, x_pad, W1)
    acc1 = _make_scatter(128)(g1, src_all, dst_all, z128)
    g2 = _t_mid(acc1, b1r, dinv, W2p, relu=True)
    acc2 = _make_scatter(128)(g2, src_all, dst_all, z128)
    g3 = _t_mid(acc2, b2r, dinv, W3p, relu=False)
    acc3 = _make_scatter(128)(g3, src_all, dst_all, z128)
    g4 = _t_mid(acc3, b3r, dinv, W4, relu=True)
    acc4 = _make_scatter(128)(g4, src_all, dst_all, z128)
    out = _t_final(acc4, b4r, dinv)
    return out[:N]


# speedup vs baseline: 2.7474x; 1.0538x over previous
import functools

import jax
import jax.numpy as jnp
from jax import lax
from jax.experimental import pallas as pl
from jax.experimental.pallas import tpu as pltpu
from jax.experimental.pallas import tpu_sc as plsc

N = 10000
D_IN = 128
D_MID = 128
D_BOT = 64

NC = 2
NS = 16
NW = NC * NS

N_PAD = 10240
ROWS_PER_TILE = N_PAD // NS

CH = 88
E_EDGES = 320000
E_ALL = E_EDGES + N
CHUNKS_PER_TILE = 120
EDGES_PER_TILE = CHUNKS_PER_TILE * CH
E_PAD = EDGES_PER_TILE * NW

BN = 1024
GRID = N_PAD // BN


NBUF = 4
NIDX = 8
GROUPS = CHUNKS_PER_TILE // 3
GROUPS6 = CHUNKS_PER_TILE // NIDX


@functools.lru_cache(maxsize=None)
def _make_deg_scatter():
    mesh = plsc.VectorSubcoreMesh(core_axis_name="c", subcore_axis_name="s")

    @functools.partial(
        pl.kernel,
        out_type=jax.ShapeDtypeStruct((NC, N_PAD, 16), jnp.float32),
        mesh=mesh,
        compiler_params=pltpu.CompilerParams(use_tc_tiling_on_sc=False),
        scratch_types=[
            pltpu.VMEM((CHUNKS_PER_TILE, CH), jnp.int32),
            pltpu.VMEM((CH, 16), jnp.float32),
            pltpu.VMEM_SHARED((N_PAD, 16), jnp.float32),
            pltpu.SemaphoreType.DMA,
            pltpu.SemaphoreType.DMA,
            pltpu.SemaphoreType.DMA,
        ],
    )
    def degk(dst_hbm, ones_hbm, zrows_hbm, out_hbm, didx, ones_v, acc,
             ss0, ss1, ss2):
        c = lax.axis_index("c")
        s = lax.axis_index("s")
        tid = c * NS + s
        sss = [ss0, ss1, ss2]

        pltpu.sync_copy(ones_hbm, ones_v)
        pltpu.sync_copy(dst_hbm.at[tid], didx)
        pltpu.sync_copy(zrows_hbm, acc.at[pl.ds(s * ROWS_PER_TILE, ROWS_PER_TILE)])
        plsc.subcore_barrier()

        def group(g, carry):
            for b in range(3):
                k = g * 3 + b
                pltpu.async_copy(ones_v, acc.at[didx.at[k]], sss[b], add=True)

                @pl.when(k >= 2)
                def _():
                    bp = (b + 1) % 3
                    pltpu.make_async_copy(
                        ones_v, acc.at[didx.at[k - 2]], sss[bp]).wait()
            return carry

        lax.fori_loop(0, GROUPS, group, 0)
        for k in (CHUNKS_PER_TILE - 2, CHUNKS_PER_TILE - 1):
            pltpu.make_async_copy(ones_v, acc.at[didx.at[k]],
                                  sss[k % 3]).wait()
        plsc.subcore_barrier()
        pltpu.sync_copy(
            acc.at[pl.ds(s * ROWS_PER_TILE, ROWS_PER_TILE)],
            out_hbm.at[c, pl.ds(s * ROWS_PER_TILE, ROWS_PER_TILE)],
        )

    return degk


@functools.lru_cache(maxsize=None)
def _make_scatter(d: int):
    mesh = plsc.VectorSubcoreMesh(core_axis_name="c", subcore_axis_name="s")

    @functools.partial(
        pl.kernel,
        out_type=jax.ShapeDtypeStruct((NC, N_PAD, d), jnp.float32),
        mesh=mesh,
        compiler_params=(None if d == 128 else
                         pltpu.CompilerParams(use_tc_tiling_on_sc=False)),
        scratch_types=[
            pltpu.VMEM((NIDX, CH), jnp.int32),
            pltpu.VMEM((NIDX, CH), jnp.int32),
            pltpu.VMEM((NBUF, CH, d), jnp.float32),
            pltpu.VMEM_SHARED((N_PAD, d), jnp.float32),
            [pltpu.SemaphoreType.DMA] * NBUF,
            [pltpu.SemaphoreType.DMA] * NBUF,
            [pltpu.SemaphoreType.DMA] * NIDX,
        ],
    )
    def scat(g_hbm, src_hbm, dst_hbm, zrows_hbm, out_hbm,
             sidx, didx, rows, acc, sgs, sss, sis):
        c = lax.axis_index("c")
        s = lax.axis_index("s")
        tid = c * NS + s

        def idx_copy(k, j, sync):
            if sync:
                pltpu.sync_copy(src_hbm.at[tid, k], sidx.at[j])
                pltpu.sync_copy(dst_hbm.at[tid, k], didx.at[j])
            else:
                pltpu.async_copy(src_hbm.at[tid, k], sidx.at[j], sis[j])
                pltpu.async_copy(dst_hbm.at[tid, k], didx.at[j], sis[j])

        def idx_wait(k, j):
            pltpu.make_async_copy(src_hbm.at[tid, k], sidx.at[j], sis[j]).wait()
            pltpu.make_async_copy(dst_hbm.at[tid, k], didx.at[j], sis[j]).wait()

        def issue_gather(j, b):
            pltpu.async_copy(g_hbm.at[sidx.at[j]], rows.at[b], sgs[b])

        idx_copy(0, 0, True)
        idx_copy(1, 1, True)
        idx_copy(2, 2, True)
        issue_gather(0, 0)
        issue_gather(1, 1)
        issue_gather(2, 2)
        idx_copy(3, 3, False)
        idx_copy(4, 4, False)
        pltpu.sync_copy(zrows_hbm, acc.at[pl.ds(s * ROWS_PER_TILE, ROWS_PER_TILE)])
        plsc.subcore_barrier()

        def group(g, carry):
            for u in range(NIDX):
                k = g * NIDX + u
                b = u % NBUF
                j = u
                jn = (u + 3) % NIDX
                jf = (u + 5) % NIDX
                bn = (b + 3) % NBUF
                pltpu.make_async_copy(
                    g_hbm.at[sidx.at[j]], rows.at[b], sgs[b]).wait()
                pltpu.async_copy(rows.at[b], acc.at[didx.at[j]], sss[b],
                                 add=True)

                jp = (u - 1) % NIDX

                @pl.when(k >= 1)
                def _():
                    pltpu.make_async_copy(
                        rows.at[bn], acc.at[didx.at[jp]], sss[bn]).wait()

                @pl.when(k + 3 < CHUNKS_PER_TILE)
                def _():
                    idx_wait(k + 3, jn)
                    issue_gather(jn, bn)

                @pl.when(k + 5 < CHUNKS_PER_TILE)
                def _():
                    idx_copy(k + 5, jf, False)
            return carry

        lax.fori_loop(0, GROUPS6, group, 0)
        kl = CHUNKS_PER_TILE - 1
        pltpu.make_async_copy(
            rows.at[kl % NBUF], acc.at[didx.at[kl % NIDX]],
            sss[kl % NBUF]).wait()
        plsc.subcore_barrier()

        pltpu.sync_copy(
            acc.at[pl.ds(s * ROWS_PER_TILE, ROWS_PER_TILE)],
            out_hbm.at[c, pl.ds(s * ROWS_PER_TILE, ROWS_PER_TILE)],
        )

    return scat


def _t_first(dacc, x_pad, w1):
    def body(dacc_ref, x_ref, w_ref, g_ref, dinv_ref):
        a = dacc_ref[...]
        deg = jnp.max(a[0] + a[1], axis=1, keepdims=True)
        dinv = lax.rsqrt(jnp.maximum(deg, 1.0))
        dinv_ref[...] = dinv
        g_ref[...] = jnp.dot(x_ref[...] * dinv, w_ref[...],
                             preferred_element_type=jnp.float32)

    return pl.pallas_call(
        body,
        grid=(GRID,),
        in_specs=[
            pl.BlockSpec((NC, BN, 16), lambda i: (0, i, 0)),
            pl.BlockSpec((BN, D_IN), lambda i: (i, 0)),
            pl.BlockSpec((D_IN, D_MID), lambda i: (0, 0)),
        ],
        out_specs=[
            pl.BlockSpec((BN, D_MID), lambda i: (i, 0)),
            pl.BlockSpec((BN, 1), lambda i: (i, 0)),
        ],
        out_shape=[
            jax.ShapeDtypeStruct((N_PAD, D_MID), jnp.float32),
            jax.ShapeDtypeStruct((N_PAD, 1), jnp.float32),
        ],
    )(dacc, x_pad, w1)


def _t_mid(acc, bias, dinv, w, relu: bool):
    dp = acc.shape[2]
    dn = w.shape[1]

    def body(acc_ref, b_ref, dinv_ref, w_ref, g_ref):
        a = acc_ref[...]
        dinv = dinv_ref[...]
        conv = (a[0] + a[1]) * dinv + b_ref[...]
        if relu:
            conv = jnp.maximum(conv, 0.0)
        g_ref[...] = jnp.dot(conv * dinv, w_ref[...],
                             preferred_element_type=jnp.float32)

    return pl.pallas_call(
        body,
        grid=(GRID,),
        in_specs=[
            pl.BlockSpec((NC, BN, dp), lambda i: (0, i, 0)),
            pl.BlockSpec((1, dp), lambda i: (0, 0)),
            pl.BlockSpec((BN, 1), lambda i: (i, 0)),
            pl.BlockSpec((dp, dn), lambda i: (0, 0)),
        ],
        out_specs=pl.BlockSpec((BN, dn), lambda i: (i, 0)),
        out_shape=jax.ShapeDtypeStruct((N_PAD, dn), jnp.float32),
    )(acc, bias, dinv, w)


def _t_final(acc, bias, dinv):
    dp = acc.shape[2]

    def body(acc_ref, b_ref, dinv_ref, o_ref):
        a = acc_ref[...]
        o_ref[...] = (a[0] + a[1]) * dinv_ref[...] + b_ref[...]

    return pl.pallas_call(
        body,
        grid=(GRID,),
        in_specs=[
            pl.BlockSpec((NC, BN, dp), lambda i: (0, i, 0)),
            pl.BlockSpec((1, dp), lambda i: (0, 0)),
            pl.BlockSpec((BN, 1), lambda i: (i, 0)),
        ],
        out_specs=pl.BlockSpec((BN, dp), lambda i: (i, 0)),
        out_shape=jax.ShapeDtypeStruct((N_PAD, dp), jnp.float32),
    )(acc, bias, dinv)


def kernel(x, edge_index, W1, b1, W2, b2, W3, b3, W4, b4):
    src = edge_index[0].astype(jnp.int32)
    dst = edge_index[1].astype(jnp.int32)
    loops = jnp.arange(N, dtype=jnp.int32)
    padv = N + jnp.arange(E_PAD - E_ALL, dtype=jnp.int32) % (N_PAD - N)
    src_all = jnp.concatenate([src, loops, padv]).reshape(NW, CHUNKS_PER_TILE, CH)
    dst_all = jnp.concatenate([dst, loops, padv]).reshape(NW, CHUNKS_PER_TILE, CH)

    x_pad = jnp.zeros((N_PAD, D_IN), jnp.float32).at[:N].set(x)
    ones128 = jnp.ones((CH, 16), jnp.float32)
    zdeg = jnp.zeros((ROWS_PER_TILE, 16), jnp.float32)
    z128 = jnp.zeros((ROWS_PER_TILE, 128), jnp.float32)

    z64 = jnp.zeros((ROWS_PER_TILE, 64), jnp.float32)

    b1r = b1.reshape(1, -1)
    b2r = b2.reshape(1, -1)
    b3r = b3.reshape(1, -1)
    b4r = b4.reshape(1, -1)

    dacc = _make_deg_scatter()(dst_all, ones128, zdeg)

    g1, dinv = _t_first(dacc, x_pad, W1)
    acc1 = _make_scatter(128)(g1, src_all, dst_all, z128)
    g2 = _t_mid(acc1, b1r, dinv, W2, relu=True)
    acc2 = _make_scatter(64)(g2, src_all, dst_all, z64)
    g3 = _t_mid(acc2, b2r, dinv, W3, relu=False)
    acc3 = _make_scatter(128)(g3, src_all, dst_all, z128)
    g4 = _t_mid(acc3, b3r, dinv, W4, relu=True)
    acc4 = _make_scatter(128)(g4, src_all, dst_all, z128)
    out = _t_final(acc4, b4r, dinv)
    return out[:N]


# speedup vs baseline: 2.8429x; 1.0347x over previous
import functools

import jax
import jax.numpy as jnp
from jax import lax
from jax.experimental import pallas as pl
from jax.experimental.pallas import tpu as pltpu
from jax.experimental.pallas import tpu_sc as plsc

N = 10000
D_IN = 128
D_MID = 128
D_BOT = 64

NC = 2
NS = 16
NW = NC * NS

N_PAD = 10240
ROWS_PER_TILE = N_PAD // NS

CH = 96
E_EDGES = 320000
E_ALL = E_EDGES + N
CHUNKS_PER_TILE = 108
EDGES_PER_TILE = CHUNKS_PER_TILE * CH
E_PAD = EDGES_PER_TILE * NW

BN = 2048
GRID = N_PAD // BN


NBUF = 3
NIDX = 6
GROUPS = CHUNKS_PER_TILE // 3
GROUPS6 = CHUNKS_PER_TILE // NIDX


@functools.lru_cache(maxsize=None)
def _make_deg_scatter():
    mesh = plsc.VectorSubcoreMesh(core_axis_name="c", subcore_axis_name="s")

    @functools.partial(
        pl.kernel,
        out_type=jax.ShapeDtypeStruct((NC, N_PAD, 16), jnp.float32),
        mesh=mesh,
        compiler_params=pltpu.CompilerParams(use_tc_tiling_on_sc=False),
        scratch_types=[
            pltpu.VMEM((CHUNKS_PER_TILE, CH), jnp.int32),
            pltpu.VMEM((CH, 16), jnp.float32),
            pltpu.VMEM_SHARED((N_PAD, 16), jnp.float32),
            pltpu.SemaphoreType.DMA,
            pltpu.SemaphoreType.DMA,
            pltpu.SemaphoreType.DMA,
        ],
    )
    def degk(dst_hbm, ones_hbm, zrows_hbm, out_hbm, didx, ones_v, acc,
             ss0, ss1, ss2):
        c = lax.axis_index("c")
        s = lax.axis_index("s")
        tid = c * NS + s
        sss = [ss0, ss1, ss2]

        pltpu.sync_copy(ones_hbm, ones_v)
        pltpu.sync_copy(dst_hbm.at[tid], didx)
        pltpu.sync_copy(zrows_hbm, acc.at[pl.ds(s * ROWS_PER_TILE, ROWS_PER_TILE)])
        plsc.subcore_barrier()

        def group(g, carry):
            for b in range(3):
                k = g * 3 + b
                pltpu.async_copy(ones_v, acc.at[didx.at[k]], sss[b], add=True)

                @pl.when(k >= 2)
                def _():
                    bp = (b + 1) % 3
                    pltpu.make_async_copy(
                        ones_v, acc.at[didx.at[k - 2]], sss[bp]).wait()
            return carry

        lax.fori_loop(0, GROUPS, group, 0)
        for k in (CHUNKS_PER_TILE - 2, CHUNKS_PER_TILE - 1):
            pltpu.make_async_copy(ones_v, acc.at[didx.at[k]],
                                  sss[k % 3]).wait()
        plsc.subcore_barrier()
        pltpu.sync_copy(
            acc.at[pl.ds(s * ROWS_PER_TILE, ROWS_PER_TILE)],
            out_hbm.at[c, pl.ds(s * ROWS_PER_TILE, ROWS_PER_TILE)],
        )

    return degk


@functools.lru_cache(maxsize=None)
def _make_scatter(d: int):
    mesh = plsc.VectorSubcoreMesh(core_axis_name="c", subcore_axis_name="s")

    @functools.partial(
        pl.kernel,
        out_type=jax.ShapeDtypeStruct((NC, N_PAD, d), jnp.float32),
        mesh=mesh,
        compiler_params=(None if d == 128 else
                         pltpu.CompilerParams(use_tc_tiling_on_sc=False)),
        scratch_types=[
            pltpu.VMEM((NIDX, CH), jnp.int32),
            pltpu.VMEM((NIDX, CH), jnp.int32),
            pltpu.VMEM((NBUF, CH, d), jnp.float32),
            pltpu.VMEM_SHARED((N_PAD, d), jnp.float32),
            [pltpu.SemaphoreType.DMA] * NBUF,
            [pltpu.SemaphoreType.DMA] * NBUF,
            [pltpu.SemaphoreType.DMA] * NIDX,
        ],
    )
    def scat(g_hbm, src_hbm, dst_hbm, zrows_hbm, out_hbm,
             sidx, didx, rows, acc, sgs, sss, sis):
        c = lax.axis_index("c")
        s = lax.axis_index("s")
        tid = c * NS + s

        def idx_copy(k, j, sync):
            if sync:
                pltpu.sync_copy(src_hbm.at[tid, k], sidx.at[j])
                pltpu.sync_copy(dst_hbm.at[tid, k], didx.at[j])
            else:
                pltpu.async_copy(src_hbm.at[tid, k], sidx.at[j], sis[j])
                pltpu.async_copy(dst_hbm.at[tid, k], didx.at[j], sis[j])

        def idx_wait(k, j):
            pltpu.make_async_copy(src_hbm.at[tid, k], sidx.at[j], sis[j]).wait()
            pltpu.make_async_copy(dst_hbm.at[tid, k], didx.at[j], sis[j]).wait()

        def issue_gather(j, b):
            pltpu.async_copy(g_hbm.at[sidx.at[j]], rows.at[b], sgs[b])

        idx_copy(0, 0, True)
        idx_copy(1, 1, True)
        issue_gather(0, 0)
        issue_gather(1, 1)
        idx_copy(2, 2, False)
        idx_copy(3, 3, False)
        pltpu.sync_copy(zrows_hbm, acc.at[pl.ds(s * ROWS_PER_TILE, ROWS_PER_TILE)])
        plsc.subcore_barrier()

        def group(g, carry):
            for u in range(NIDX):
                k = g * NIDX + u
                b = u % NBUF
                j = u
                jn = (u + 2) % NIDX
                jf = (u + 4) % NIDX
                bn = (b + 2) % NBUF
                pltpu.make_async_copy(
                    g_hbm.at[sidx.at[j]], rows.at[b], sgs[b]).wait()
                pltpu.async_copy(rows.at[b], acc.at[didx.at[j]], sss[b],
                                 add=True)

                jp = (u - 1) % NIDX

                @pl.when(k >= 1)
                def _():
                    pltpu.make_async_copy(
                        rows.at[bn], acc.at[didx.at[jp]], sss[bn]).wait()

                @pl.when(k + 2 < CHUNKS_PER_TILE)
                def _():
                    idx_wait(k + 2, jn)
                    issue_gather(jn, bn)

                @pl.when(k + 4 < CHUNKS_PER_TILE)
                def _():
                    idx_copy(k + 4, jf, False)
            return carry

        lax.fori_loop(0, GROUPS6, group, 0)
        kl = CHUNKS_PER_TILE - 1
        pltpu.make_async_copy(
            rows.at[kl % NBUF], acc.at[didx.at[kl % NIDX]],
            sss[kl % NBUF]).wait()
        plsc.subcore_barrier()

        pltpu.sync_copy(
            acc.at[pl.ds(s * ROWS_PER_TILE, ROWS_PER_TILE)],
            out_hbm.at[c, pl.ds(s * ROWS_PER_TILE, ROWS_PER_TILE)],
        )

    return scat


def _t_first(dacc, x_pad, w1):
    def body(dacc_ref, x_ref, w_ref, g_ref, dinv_ref):
        a = dacc_ref[...]
        deg = jnp.max(a[0] + a[1], axis=1, keepdims=True)
        dinv = lax.rsqrt(jnp.maximum(deg, 1.0))
        dinv_ref[...] = dinv
        g_ref[...] = jnp.dot(x_ref[...] * dinv, w_ref[...],
                             preferred_element_type=jnp.float32)

    return pl.pallas_call(
        body,
        grid=(GRID,),
        in_specs=[
            pl.BlockSpec((NC, BN, 16), lambda i: (0, i, 0)),
            pl.BlockSpec((BN, D_IN), lambda i: (i, 0)),
            pl.BlockSpec((D_IN, D_MID), lambda i: (0, 0)),
        ],
        out_specs=[
            pl.BlockSpec((BN, D_MID), lambda i: (i, 0)),
            pl.BlockSpec((BN, 1), lambda i: (i, 0)),
        ],
        out_shape=[
            jax.ShapeDtypeStruct((N_PAD, D_MID), jnp.float32),
            jax.ShapeDtypeStruct((N_PAD, 1), jnp.float32),
        ],
    )(dacc, x_pad, w1)


def _t_mid(acc, bias, dinv, w, relu: bool):
    dp = acc.shape[2]
    dn = w.shape[1]

    def body(acc_ref, b_ref, dinv_ref, w_ref, g_ref):
        a = acc_ref[...]
        dinv = dinv_ref[...]
        conv = (a[0] + a[1]) * dinv + b_ref[...]
        if relu:
            conv = jnp.maximum(conv, 0.0)
        g_ref[...] = jnp.dot(conv * dinv, w_ref[...],
                             preferred_element_type=jnp.float32)

    return pl.pallas_call(
        body,
        grid=(GRID,),
        in_specs=[
            pl.BlockSpec((NC, BN, dp), lambda i: (0, i, 0)),
            pl.BlockSpec((1, dp), lambda i: (0, 0)),
            pl.BlockSpec((BN, 1), lambda i: (i, 0)),
            pl.BlockSpec((dp, dn), lambda i: (0, 0)),
        ],
        out_specs=pl.BlockSpec((BN, dn), lambda i: (i, 0)),
        out_shape=jax.ShapeDtypeStruct((N_PAD, dn), jnp.float32),
    )(acc, bias, dinv, w)


def _t_final(acc, bias, dinv):
    dp = acc.shape[2]

    def body(acc_ref, b_ref, dinv_ref, o_ref):
        a = acc_ref[...]
        o_ref[...] = (a[0] + a[1]) * dinv_ref[...] + b_ref[...]

    return pl.pallas_call(
        body,
        grid=(GRID,),
        in_specs=[
            pl.BlockSpec((NC, BN, dp), lambda i: (0, i, 0)),
            pl.BlockSpec((1, dp), lambda i: (0, 0)),
            pl.BlockSpec((BN, 1), lambda i: (i, 0)),
        ],
        out_specs=pl.BlockSpec((BN, dp), lambda i: (i, 0)),
        out_shape=jax.ShapeDtypeStruct((N_PAD, dp), jnp.float32),
    )(acc, bias, dinv)


def kernel(x, edge_index, W1, b1, W2, b2, W3, b3, W4, b4):
    src = edge_index[0].astype(jnp.int32)
    dst = edge_index[1].astype(jnp.int32)
    loops = jnp.arange(N, dtype=jnp.int32)
    padv = N + jnp.arange(E_PAD - E_ALL, dtype=jnp.int32) % (N_PAD - N)
    src_all = jnp.concatenate([src, loops, padv]).reshape(NW, CHUNKS_PER_TILE, CH)
    dst_all = jnp.concatenate([dst, loops, padv]).reshape(NW, CHUNKS_PER_TILE, CH)

    x_pad = jnp.zeros((N_PAD, D_IN), jnp.float32).at[:N].set(x)
    ones128 = jnp.ones((CH, 16), jnp.float32)
    zdeg = jnp.zeros((ROWS_PER_TILE, 16), jnp.float32)
    z128 = jnp.zeros((ROWS_PER_TILE, 128), jnp.float32)

    z64 = jnp.zeros((ROWS_PER_TILE, 64), jnp.float32)

    b1r = b1.reshape(1, -1)
    b2r = b2.reshape(1, -1)
    b3r = b3.reshape(1, -1)
    b4r = b4.reshape(1, -1)

    dacc = _make_deg_scatter()(dst_all, ones128, zdeg)

    g1, dinv = _t_first(dacc, x_pad, W1)
    acc1 = _make_scatter(128)(g1, src_all, dst_all, z128)
    g2 = _t_mid(acc1, b1r, dinv, W2, relu=True)
    acc2 = _make_scatter(64)(g2, src_all, dst_all, z64)
    g3 = _t_mid(acc2, b2r, dinv, W3, relu=False)
    acc3 = _make_scatter(128)(g3, src_all, dst_all, z128)
    g4 = _t_mid(acc3, b3r, dinv, W4, relu=True)
    acc4 = _make_scatter(128)(g4, src_all, dst_all, z128)
    out = _t_final(acc4, b4r, dinv)
    return out[:N]
